# baseline (speedup 1.0000x reference)
import functools

import jax
import jax.numpy as jnp
from jax import lax
from jax.experimental import pallas as pl
from jax.experimental.pallas import tpu as pltpu
from jax.experimental.pallas import tpu_sc as plsc

_N = 16384
_D = 64
_K = 1024
_BLK = 2048

_NC = 2
_NS = 16
_NW = _NC * _NS
_BPW = _N // _NW


def _tc_body(x_ref, e_ref, ind_ref, diff_ref):
    x = x_ref[...]
    e = e_ref[...]
    xsq = jnp.sum(x * x, axis=1, keepdims=True)
    esq = jnp.sum(e * e, axis=0, keepdims=True)
    xe = jax.lax.dot_general(
        x, e, (((1,), (0,)), ((), ())),
        preferred_element_type=jnp.float32)
    neg = -(xsq - 2.0 * xe + esq)
    m = jnp.max(neg, axis=1, keepdims=True)
    iota = jax.lax.broadcasted_iota(jnp.int32, neg.shape, 1)
    ind_ref[...] = jnp.min(jnp.where(neg == m, iota, _K), axis=1)

    @pl.when(pl.program_id(0) == 0)
    def _():
        diff_ref[0, 0] = 0.0

    diff_ref[0, 0] += -jnp.sum(m)


_mesh = plsc.VectorSubcoreMesh(core_axis_name="c", subcore_axis_name="s")
_CHUNK = 128
_NCHUNK = _BPW // _CHUNK


@functools.partial(
    pl.kernel, mesh=_mesh,
    out_type=jax.ShapeDtypeStruct((_NW, _BPW, _D), jnp.float32),
    scratch_types=[
        pltpu.VMEM((_NCHUNK, _CHUNK), jnp.int32),
        pltpu.VMEM((_BPW, _D), jnp.float32),
        pltpu.SemaphoreType.DMA,
    ],
    compiler_params=pltpu.CompilerParams(use_tc_tiling_on_sc=False),
)
def _sc_gather(tab_hbm, idx_hbm, out_hbm, idx_v, rows_v, sem):
    wid = lax.axis_index("s") * _NC + lax.axis_index("c")
    pltpu.sync_copy(idx_hbm.at[wid], idx_v)
    copies = [
        pltpu.async_copy(tab_hbm.at[idx_v.at[j]],
                         rows_v.at[pl.ds(j * _CHUNK, _CHUNK)], sem)
        for j in range(_NCHUNK)
    ]
    for c in copies:
        c.wait()
    pltpu.sync_copy(rows_v, out_hbm.at[wid])


@jax.jit
def kernel(input, embed):
    flat = input.reshape(-1, _D)
    ind, diff = pl.pallas_call(
        _tc_body,
        grid=(_N // _BLK,),
        in_specs=[
            pl.BlockSpec((_BLK, _D), lambda i: (i, 0)),
            pl.BlockSpec((_D, _K), lambda i: (0, 0)),
        ],
        out_specs=[
            pl.BlockSpec((_BLK,), lambda i: (i,)),
            pl.BlockSpec(memory_space=pltpu.SMEM, block_shape=(1, 1),
                         index_map=lambda i: (0, 0)),
        ],
        out_shape=[
            jax.ShapeDtypeStruct((_N,), jnp.int32),
            jax.ShapeDtypeStruct((1, 1), jnp.float32),
        ],
        compiler_params=pltpu.CompilerParams(
            dimension_semantics=("arbitrary",)),
    )(flat, embed)
    quantize = _sc_gather(
        embed.T, ind.reshape(_NW, _NCHUNK, _CHUNK)).reshape(input.shape)
    embed_ind = ind.reshape(input.shape[:-1])
    return quantize, diff[0, 0] / float(_N * _D), embed_ind


# speedup vs baseline: 1.0514x; 1.0514x over previous
import functools

import jax
import jax.numpy as jnp
from jax import lax
from jax.experimental import pallas as pl
from jax.experimental.pallas import tpu as pltpu
from jax.experimental.pallas import tpu_sc as plsc

_N = 16384
_D = 64
_K = 1024
_BLK = 2048

_NC = 2
_NS = 16
_NW = _NC * _NS
_BPW = _N // _NW


def _tc_body(x_ref, e_ref, ind_ref, diff_ref):
    x = x_ref[...]
    e = e_ref[...]
    xsq = jnp.sum(x * x, axis=1, keepdims=True)
    esq = jnp.sum(e * e, axis=0, keepdims=True)
    xe = jax.lax.dot_general(
        x, e, (((1,), (0,)), ((), ())),
        preferred_element_type=jnp.float32)
    neg = (xe + xe - xsq) - esq
    m = jnp.max(neg, axis=1, keepdims=True)
    niota = -jax.lax.broadcasted_iota(
        jnp.int32, neg.shape, 1).astype(jnp.float32)
    picked = jnp.max(jnp.where(neg == m, niota, -jnp.inf), axis=1)
    ind_ref[...] = (-picked).astype(jnp.int32)

    @pl.when(pl.program_id(0) == 0)
    def _():
        diff_ref[0, 0] = 0.0

    diff_ref[0, 0] += -jnp.sum(m)


_mesh = plsc.VectorSubcoreMesh(core_axis_name="c", subcore_axis_name="s")
_CHUNK = 128
_NCHUNK = _BPW // _CHUNK


@functools.partial(
    pl.kernel, mesh=_mesh,
    out_type=jax.ShapeDtypeStruct((_NW, _BPW, _D), jnp.float32),
    scratch_types=[
        pltpu.VMEM((_NCHUNK, _CHUNK), jnp.int32),
        pltpu.VMEM((_BPW, _D), jnp.float32),
        pltpu.SemaphoreType.DMA,
    ],
    compiler_params=pltpu.CompilerParams(use_tc_tiling_on_sc=False),
)
def _sc_gather(tab_hbm, idx_hbm, out_hbm, idx_v, rows_v, sem):
    wid = lax.axis_index("s") * _NC + lax.axis_index("c")
    pltpu.sync_copy(idx_hbm.at[wid], idx_v)
    copies = [
        pltpu.async_copy(tab_hbm.at[idx_v.at[j]],
                         rows_v.at[pl.ds(j * _CHUNK, _CHUNK)], sem)
        for j in range(_NCHUNK)
    ]
    for c in copies:
        c.wait()
    pltpu.sync_copy(rows_v, out_hbm.at[wid])


@jax.jit
def kernel(input, embed):
    flat = input.reshape(-1, _D)
    ind, diff = pl.pallas_call(
        _tc_body,
        grid=(_N // _BLK,),
        in_specs=[
            pl.BlockSpec((_BLK, _D), lambda i: (i, 0)),
            pl.BlockSpec((_D, _K), lambda i: (0, 0)),
        ],
        out_specs=[
            pl.BlockSpec((_BLK,), lambda i: (i,)),
            pl.BlockSpec(memory_space=pltpu.SMEM, block_shape=(1, 1),
                         index_map=lambda i: (0, 0)),
        ],
        out_shape=[
            jax.ShapeDtypeStruct((_N,), jnp.int32),
            jax.ShapeDtypeStruct((1, 1), jnp.float32),
        ],
        compiler_params=pltpu.CompilerParams(
            dimension_semantics=("arbitrary",)),
    )(flat, embed)
    quantize = _sc_gather(
        embed.T, ind.reshape(_NW, _NCHUNK, _CHUNK)).reshape(input.shape)
    embed_ind = ind.reshape(input.shape[:-1])
    return quantize, diff[0, 0] / float(_N * _D), embed_ind


# speedup vs baseline: 1.9814x; 1.8845x over previous
import jax
import jax.numpy as jnp
from jax.experimental import pallas as pl
from jax.experimental.pallas import tpu as pltpu

_N = 16384
_D = 64
_K = 1024
_BLK = 2048


def _vq_body(x_ref, e_ref, q_ref, ind_ref, diff_ref):
    x = x_ref[...]
    e = e_ref[...]
    xsq = jnp.sum(x * x, axis=1, keepdims=True)
    esq = jnp.sum(e * e, axis=0, keepdims=True)
    xe = jax.lax.dot_general(
        x, e, (((1,), (0,)), ((), ())),
        preferred_element_type=jnp.float32)
    neg = (xe + xe - xsq) - esq
    m = jnp.max(neg, axis=1, keepdims=True)
    niota = -jax.lax.broadcasted_iota(
        jnp.int32, neg.shape, 1).astype(jnp.float32)
    eq = neg == m
    picked = jnp.max(jnp.where(eq, niota, -jnp.inf), axis=1)
    ind_ref[...] = (-picked).astype(jnp.int32)
    onehot = (niota == picked[:, None]).astype(jnp.float32)
    q_ref[...] = jax.lax.dot_general(
        onehot, e, (((1,), (1,)), ((), ())),
        preferred_element_type=jnp.float32)

    @pl.when(pl.program_id(0) == 0)
    def _():
        diff_ref[0, 0] = 0.0

    diff_ref[0, 0] += -jnp.sum(m)


@jax.jit
def kernel(input, embed):
    flat = input.reshape(-1, _D)
    q, ind, diff = pl.pallas_call(
        _vq_body,
        grid=(_N // _BLK,),
        in_specs=[
            pl.BlockSpec((_BLK, _D), lambda i: (i, 0)),
            pl.BlockSpec((_D, _K), lambda i: (0, 0)),
        ],
        out_specs=[
            pl.BlockSpec((_BLK, _D), lambda i: (i, 0)),
            pl.BlockSpec((_BLK,), lambda i: (i,)),
            pl.BlockSpec(memory_space=pltpu.SMEM, block_shape=(1, 1),
                         index_map=lambda i: (0, 0)),
        ],
        out_shape=[
            jax.ShapeDtypeStruct((_N, _D), jnp.float32),
            jax.ShapeDtypeStruct((_N,), jnp.int32),
            jax.ShapeDtypeStruct((1, 1), jnp.float32),
        ],
        compiler_params=pltpu.CompilerParams(
            dimension_semantics=("arbitrary",)),
    )(flat, embed)
    quantize = q.reshape(input.shape)
    embed_ind = ind.reshape(input.shape[:-1])
    return quantize, diff[0, 0] / float(_N * _D), embed_ind
